# baseline (speedup 1.0000x reference)
import functools

import jax
import jax.numpy as jnp
from jax import lax
from jax.experimental import pallas as pl
from jax.experimental.pallas import tpu as pltpu
from jax.experimental.pallas import tpu_sc as plsc

EMBED_DIM = 32
LANES = 16


def _cause_sc(batch: int):
    info = plsc.get_sparse_core_info()
    nc, ns = info.num_cores, info.num_subcores
    nw = nc * ns
    assert batch % (8 * nw) == 0
    bpw = batch // nw
    groups = bpw // LANES

    mesh = plsc.VectorSubcoreMesh(core_axis_name="c", subcore_axis_name="s")

    @functools.partial(
        pl.kernel,
        mesh=mesh,
        out_type=jax.ShapeDtypeStruct((batch,), jnp.float32),
        scratch_types=[
            pltpu.VMEM((bpw,), jnp.int32),
            pltpu.VMEM((bpw,), jnp.int32),
            pltpu.VMEM((bpw, EMBED_DIM), jnp.float32),
            pltpu.VMEM((bpw, EMBED_DIM), jnp.float32),
            pltpu.VMEM((bpw,), jnp.float32),
            pltpu.VMEM((bpw,), jnp.float32),
            pltpu.VMEM((bpw,), jnp.float32),
            pltpu.SemaphoreType.DMA,
        ],
        compiler_params=pltpu.CompilerParams(
            needs_layout_passes=False, use_tc_tiling_on_sc=False
        ),
    )
    def k(user_hbm, item_hbm, ue_hbm, ie_hbm, ub_hbm, ib_hbm, out_hbm,
          uidx_v, iidx_v, urows_v, irows_v, ub_v, ib_v, out_v, sem):
        wid = lax.axis_index("s") * nc + lax.axis_index("c")
        base = wid * bpw

        pltpu.sync_copy(user_hbm.at[pl.ds(base, bpw)], uidx_v)
        pltpu.sync_copy(item_hbm.at[pl.ds(base, bpw)], iidx_v)

        c1 = pltpu.async_copy(ue_hbm.at[uidx_v], urows_v, sem)
        c2 = pltpu.async_copy(ie_hbm.at[iidx_v], irows_v, sem)
        c3 = pltpu.async_copy(ub_hbm.at[uidx_v], ub_v, sem)
        c4 = pltpu.async_copy(ib_hbm.at[iidx_v], ib_v, sem)
        c1.wait()
        c2.wait()
        c3.wait()
        c4.wait()

        def body(g, carry):
            rows = g * LANES + lax.iota(jnp.int32, LANES)
            flat = rows * EMBED_DIM
            acc = ub_v[pl.ds(g * LANES, LANES)] + ib_v[pl.ds(g * LANES, LANES)]
            for d in range(EMBED_DIM):
                col = jnp.full((LANES,), d, jnp.int32)
                u = plsc.load_gather(urows_v, [rows, col])
                i = plsc.load_gather(irows_v, [rows, col])
                acc = acc + u * i
            out_v[pl.ds(g * LANES, LANES)] = acc
            return carry

        lax.fori_loop(0, groups, body, 0)
        pltpu.sync_copy(out_v, out_hbm.at[pl.ds(base, bpw)])

    return k


def kernel(user, item, user_e, item_e_c, user_b, item_b):
    batch = user.shape[0]
    f = _cause_sc(batch)
    return f(
        user.astype(jnp.int32),
        item.astype(jnp.int32),
        user_e,
        item_e_c,
        user_b.reshape(-1),
        item_b.reshape(-1),
    )
